# baseline (speedup 1.0000x reference)
import functools

import jax
import jax.numpy as jnp
from jax import lax
from jax.experimental import pallas as pl
from jax.experimental.pallas import tpu as pltpu
from jax.experimental.pallas import tpu_sc as plsc

B = 16384
C = 5
D = 64

NC = 2
NS = 16
NW = NC * NS
BPW = B // NW
G = 128
NQ = 4
QB = BPW // NQ
GB = 16


def _hsum_all_lanes(v, perms):
    for p in perms:
        v = v + jnp.take_along_axis(v, p, axis=0, mode="promise_in_bounds")
    return v


def _w2v_body(tgt_idx_hbm, ctx_idx_hbm, ttab_hbm, ctab_hbm, out_hbm,
              tidx_v, cidx_v, w_rows, c_rows, out_v, sem):
    wid = lax.axis_index("s") * NC + lax.axis_index("c")
    iota = lax.iota(jnp.int32, 16)
    perms = [iota ^ sh for sh in (8, 4, 2, 1)]
    lane_masks = [iota == j for j in range(16)]

    pltpu.sync_copy(tgt_idx_hbm.at[wid], tidx_v)
    pltpu.sync_copy(ctx_idx_hbm.at[wid], cidx_v)

    wd = [pltpu.async_copy(ttab_hbm.at[tidx_v.at[g]],
                           w_rows.at[pl.ds(g * G, G)], sem)
          for g in range(BPW // G)]
    for d in wd:
        d.wait()

    gpq = QB * C // G
    for q in range(NQ):
        cd = [pltpu.async_copy(ctab_hbm.at[cidx_v.at[q * gpq + g]],
                               c_rows.at[pl.ds(g * G, G)], sem)
              for g in range(gpq)]
        for d in cd:
            d.wait()

        def gbody(g, carry, q=q):
            b0 = g * GB
            res = [jnp.zeros((16,), jnp.float32) for _ in range(C)]
            for j in range(GB):
                bq = b0 + j
                gi = q * QB + bq
                w = [w_rows[gi, pl.ds(k * 16, 16)] for k in range(4)]
                for c in range(C):
                    p = bq * C + c
                    acc = w[0] * c_rows[p, pl.ds(0, 16)]
                    for k in range(1, 4):
                        acc = acc + w[k] * c_rows[p, pl.ds(k * 16, 16)]
                    s = _hsum_all_lanes(acc, perms)
                    fp = j * C + c
                    res[fp // 16] = jnp.where(lane_masks[fp % 16], s,
                                              res[fp // 16])
            ob = (q * QB + b0) * C
            for v in range(C):
                out_v[pl.ds(ob + v * 16, 16)] = res[v]
            return carry

        lax.fori_loop(0, QB // GB, gbody, 0)

    pltpu.sync_copy(out_v, out_hbm.at[pl.ds(wid * BPW * C, BPW * C)])


@jax.jit
def _w2v(target3d, ctx3d, target_table, context_table):
    mesh = plsc.VectorSubcoreMesh(core_axis_name="c", subcore_axis_name="s")
    k = functools.partial(
        pl.kernel,
        mesh=mesh,
        out_type=jax.ShapeDtypeStruct((B * C,), jnp.float32),
        scratch_types=[
            pltpu.VMEM((BPW // G, G), jnp.int32),
            pltpu.VMEM((BPW * C // G, G), jnp.int32),
            pltpu.VMEM((BPW, D), jnp.float32),
            pltpu.VMEM((QB * C, D), jnp.float32),
            pltpu.VMEM((BPW * C,), jnp.float32),
            pltpu.SemaphoreType.DMA,
        ],
        compiler_params=pltpu.CompilerParams(use_tc_tiling_on_sc=False),
    )(_w2v_body)
    return k(target3d, ctx3d, target_table, context_table)


def kernel(target, context, target_table, context_table):
    target3d = target.reshape(NW, BPW // G, G)
    ctx3d = context.reshape(NW, BPW * C // G, G)
    return _w2v(target3d, ctx3d, target_table, context_table).reshape(B, C)
